# baseline (speedup 1.0000x reference)
import functools

import jax
import jax.numpy as jnp
from jax import lax
from jax.experimental import pallas as pl
from jax.experimental.pallas import tpu as pltpu
from jax.experimental.pallas import tpu_sc as plsc

B, S, H = 4, 4096, 2048
K = S // 2
TS = 512
NJ = S // TS
L = 16
NV = S // L


def _tc_body(x_ref, w_ref, logits_ref):
    b = pl.program_id(0)
    j = pl.program_id(1)
    xt = x_ref[0].astype(jnp.bfloat16)
    lt = lax.dot_general(xt, w_ref[...].astype(jnp.bfloat16),
                         dimension_numbers=(((1,), (0,)), ((), ())),
                         preferred_element_type=jnp.float32)
    start = pl.multiple_of(j * TS, TS)
    logits_ref[pl.ds(b, 1), pl.ds(start, TS)] = lt[:, 0][None, :]


def _matvec(x, w):
    w2 = w.reshape(H, 1)
    return pl.pallas_call(
        _tc_body,
        grid=(B, NJ),
        in_specs=[
            pl.BlockSpec((1, TS, H), lambda b, j: (b, j, 0)),
            pl.BlockSpec((H, 1), lambda b, j: (0, 0)),
        ],
        out_specs=pl.BlockSpec((B, S), lambda b, j: (0, 0)),
        out_shape=jax.ShapeDtypeStruct((B, S), jnp.float32),
    )(x, w2)


def _lane_extract(v, t):
    lane = lax.iota(jnp.int32, L)
    return jnp.max(jnp.where(lane == t, v, jnp.int32(-2147483647 - 1)))


def _sc_route_body(logits_hbm, mask_hbm, row_v, key_v, hist_v, sem):
    nc = 2
    wid = lax.axis_index("s") * nc + lax.axis_index("c")
    i32_min = jnp.int32(-2147483647 - 1)

    @pl.when(wid < B)
    def _():
        pltpu.sync_copy(logits_hbm.at[wid], row_v)

        def conv(i, carry):
            f = row_v[pl.ds(i * L, L)]
            f = jnp.where(f == jnp.float32(0), jnp.float32(0), f)
            bits = plsc.bitcast(f, jnp.int32)
            ik = jnp.where(bits < 0,
                           jnp.bitwise_xor(jnp.bitwise_not(bits), i32_min),
                           bits)
            key_v[pl.ds(i * L, L)] = jnp.bitwise_xor(ik, i32_min)
            return carry

        lax.fori_loop(0, NV, conv, jnp.int32(0))

        prefix = jnp.int32(0)
        k_rem = jnp.int32(K)
        for level in range(4):
            shift = 24 - 8 * level

            def zhist(i, carry):
                hist_v[pl.ds(i * L, L)] = jnp.zeros((L,), jnp.int32)
                return carry

            lax.fori_loop(0, 256 // L, zhist, jnp.int32(0))

            def hpass(i, carry):
                ub = key_v[pl.ds(i * L, L)]
                digit = jnp.bitwise_and(
                    lax.shift_right_logical(ub, shift), jnp.int32(255))
                if level == 0:
                    plsc.addupdate_scatter(hist_v, [digit],
                                           jnp.ones((L,), jnp.int32))
                else:
                    hi = lax.shift_right_logical(ub, shift + 8)
                    want = lax.shift_right_logical(prefix, shift + 8)
                    plsc.addupdate_scatter(hist_v, [digit],
                                           jnp.ones((L,), jnp.int32),
                                           mask=hi == want)
                return carry

            lax.fori_loop(0, NV, hpass, jnp.int32(0))

            def scan_chunk(c, carry):
                cum, found, dstar, sub = carry
                start = 256 - L - c * L
                chunk = hist_v[pl.ds(start, L)]
                rchunk = lax.rev(chunk, (0,))
                cs = plsc.cumsum(rchunk) + cum
                hit = cs >= k_rem
                anyhit = jnp.max(plsc.all_reduce_population_count(hit))
                got = jnp.where(found == 0, anyhit, jnp.int32(0))
                tstar = jnp.max(plsc.all_reduce_ffs(hit))
                d_here = jnp.int32(start + L - 1) - tstar
                cge = _lane_extract(cs, tstar)
                hd = _lane_extract(rchunk, tstar)
                dstar = jnp.where(got > 0, d_here, dstar)
                sub = jnp.where(got > 0, cge - hd, sub)
                found = jnp.where(got > 0, jnp.int32(1), found)
                cum = cum + jnp.sum(chunk)
                return cum, found, dstar, sub

            _, _, dstar, sub = lax.fori_loop(
                0, 256 // L, scan_chunk,
                (jnp.int32(0), jnp.int32(0), jnp.int32(0), jnp.int32(0)))
            prefix = jnp.bitwise_or(prefix, lax.shift_left(dstar, shift))
            k_rem = k_rem - sub

        thr_ik = jnp.bitwise_xor(prefix, i32_min)

        def mpass(i, taken):
            ub = key_v[pl.ds(i * L, L)]
            ik = jnp.bitwise_xor(ub, i32_min)
            gt = ik > thr_ik
            eq = ub == prefix
            cs = plsc.cumsum(jnp.where(eq, jnp.int32(1), jnp.int32(0)))
            sel = eq & ((cs + taken) <= k_rem)
            row_v[pl.ds(i * L, L)] = jnp.where(
                gt | sel, jnp.float32(1), jnp.float32(0))
            return taken + jnp.max(cs)

        lax.fori_loop(0, NV, mpass, jnp.int32(0))
        pltpu.sync_copy(row_v, mask_hbm.at[wid])


def _sc_route(logits):
    mesh = plsc.VectorSubcoreMesh(core_axis_name="c", subcore_axis_name="s")
    f = functools.partial(
        pl.kernel,
        mesh=mesh,
        compiler_params=pltpu.CompilerParams(needs_layout_passes=False),
        out_type=jax.ShapeDtypeStruct((B, S), jnp.float32),
        scratch_types=[
            pltpu.VMEM((S,), jnp.float32),
            pltpu.VMEM((S,), jnp.int32),
            pltpu.VMEM((256,), jnp.int32),
            pltpu.SemaphoreType.DMA,
        ],
    )(_sc_route_body)
    return f(logits)


def kernel(x, w):
    logits = _matvec(x, w)
    mask = _sc_route(logits)
    return (mask[..., None], mask, logits)
